# baseline (speedup 1.0000x reference)
import functools

import jax
import jax.numpy as jnp
from jax import lax
from jax.experimental import pallas as pl
from jax.experimental.pallas import tpu as pltpu
from jax.experimental.pallas import tpu_sc as plsc

_NC = 2
_NS = 16
_NW = _NC * _NS
_ROW = 128
_CHUNK = 128


@functools.lru_cache(maxsize=None)
def _make_gather(vocab, dim, n_idx):
    n_per_w = n_idx // _NW
    nt = n_per_w // _CHUNK
    assert n_idx % (_NW * _CHUNK) == 0 and nt % 2 == 0 and dim <= _ROW
    mesh = plsc.VectorSubcoreMesh(core_axis_name="c", subcore_axis_name="s")

    @functools.partial(
        pl.kernel,
        out_type=jax.ShapeDtypeStruct((n_idx, _ROW), jnp.float32),
        mesh=mesh,
        scratch_types=[
            pltpu.VMEM((nt, _CHUNK), jnp.int32),
            pltpu.VMEM((2, _CHUNK, _ROW), jnp.float32),
            pltpu.SemaphoreType.DMA,
            pltpu.SemaphoreType.DMA,
        ],
        compiler_params=pltpu.CompilerParams(
            needs_layout_passes=False, disable_bounds_checks=True
        ),
    )
    def gather_kernel(idx_hbm, table_hbm, out_hbm, idx_v, rows_v, gsem, wsem):
        wid = lax.axis_index("s") * _NC + lax.axis_index("c")
        pltpu.sync_copy(idx_hbm.at[pl.ds(wid * nt, nt)], idx_v)

        pltpu.async_copy(table_hbm.at[idx_v.at[0]], rows_v.at[0], gsem)

        @pl.loop(0, nt, step=2)
        def _task(t0):
            for h in range(2):
                t = t0 + h

                @pl.when(t > 0)
                def _():
                    pltpu.make_async_copy(
                        rows_v.at[1 - h], out_hbm.at[pl.ds(0, _CHUNK)], wsem
                    ).wait()

                @pl.when(t + 1 < nt)
                def _():
                    pltpu.async_copy(
                        table_hbm.at[idx_v.at[t + 1]], rows_v.at[1 - h], gsem
                    )

                pltpu.make_async_copy(
                    table_hbm.at[idx_v.at[0]], rows_v.at[h], gsem
                ).wait()

                pltpu.async_copy(
                    rows_v.at[h],
                    out_hbm.at[pl.ds(wid * n_per_w + t * _CHUNK, _CHUNK)],
                    wsem,
                )

        pltpu.make_async_copy(rows_v.at[0], out_hbm.at[pl.ds(0, _CHUNK)], wsem).wait()

    return gather_kernel


def kernel(x, embeddings):
    batch, seq = x.shape
    vocab, dim = embeddings.shape
    n_idx = batch * seq
    padded = jnp.pad(embeddings, ((0, 0), (0, _ROW - dim)))
    idx = x.reshape(n_idx // _CHUNK, _CHUNK).astype(jnp.int32)
    out2 = _make_gather(vocab, dim, n_idx)(idx, padded)
    return out2[:, :dim].reshape(batch, seq, dim)
